# Initial kernel scaffold; baseline (speedup 1.0000x reference)
import functools

import jax
import jax.numpy as jnp
from jax import lax
from jax.experimental import pallas as pl
from jax.experimental.pallas import tpu as pltpu
from jax.experimental.pallas import tpu_sc as plsc

N = 160000
ENC = 128
D_IN = 3 * ENC


_ROWS = 640
_GRID = N // _ROWS
_WCOLS = 256


def _scored_update_body(cand_ref, graph_ref, w1_ref, w2_ref, b_ref, out_ref):
    x1 = cand_ref[...].astype(jnp.bfloat16)
    x2 = graph_ref[...].astype(jnp.bfloat16)
    w1 = w1_ref[...]
    w2 = w2_ref[...]
    y = (jnp.dot(x1, w1, preferred_element_type=jnp.float32)
         + jnp.dot(x2, w2, preferred_element_type=jnp.float32)
         + b_ref[...])
    u = y[:, :ENC]
    s = y[:, ENC:ENC + 1]
    out_ref[...] = u * s


def _scored_update(candidate_rep, graph_rep, Wu, bu, Ws, bs):
    w = jnp.zeros((D_IN, _WCOLS), jnp.float32)
    w = w.at[:, :ENC].set(Wu).at[:, ENC].set(Ws[:, 0])
    w = w.astype(jnp.bfloat16)
    b = jnp.zeros((1, _WCOLS), jnp.float32)
    b = b.at[0, :ENC].set(bu).at[0, ENC].set(bs[0])
    return pl.pallas_call(
        _scored_update_body,
        grid=(_GRID,),
        in_specs=[
            pl.BlockSpec((_ROWS, ENC), lambda i: (i, 0)),
            pl.BlockSpec((_ROWS, 2 * ENC), lambda i: (i, 0)),
            pl.BlockSpec((ENC, _WCOLS), lambda i: (0, 0)),
            pl.BlockSpec((2 * ENC, _WCOLS), lambda i: (0, 0)),
            pl.BlockSpec((1, _WCOLS), lambda i: (0, 0)),
        ],
        out_specs=pl.BlockSpec((_ROWS, ENC), lambda i: (i, 0)),
        out_shape=jax.ShapeDtypeStruct((N, ENC), jnp.float32),
    )(candidate_rep, graph_rep, w[:ENC], w[ENC:], b)


_NSC = 2
_NTEC = 16
_CH = 16000
_NCHUNK = N // _CH
_GUARD = 16
_PER_TEC = N // _NTEC
_SCAN_IT = _PER_TEC // 16
_BATCH = 128
_BUF = _PER_TEC + 2 * _BATCH
_ROWS_PER_TEC = _CH // _NTEC


@functools.partial(
    pl.kernel,
    out_type=jax.ShapeDtypeStruct((N, ENC), jnp.float32),
    mesh=plsc.VectorSubcoreMesh(core_axis_name="c", subcore_axis_name="s"),
    scratch_types=[
        pltpu.VMEM((_PER_TEC,), jnp.int32),
        pltpu.VMEM((_BUF,), jnp.int32),
        pltpu.VMEM((_BUF,), jnp.int32),
        pltpu.VMEM((_BATCH,), jnp.int32),
        pltpu.VMEM((_BATCH,), jnp.int32),
        pltpu.VMEM((_BATCH, ENC), jnp.float32),
        pltpu.VMEM_SHARED((_CH + _GUARD, ENC), jnp.float32),
        pltpu.SemaphoreType.DMA,
    ],
)
def _scatter_add(cand_hbm, idx_hbm, upd_hbm, out_hbm,
                 idx_v, pos_flat, rel_flat, pos_b, rel_b, rows_v, chunk_sp,
                 sem):
    c = lax.axis_index("c")
    s = lax.axis_index("s")
    lane = lax.iota(jnp.int32, 16)
    idx_base = s * _PER_TEC

    pltpu.sync_copy(idx_hbm.at[pl.ds(idx_base, _PER_TEC)], idx_v)

    for k in range(_NCHUNK // _NSC):
        chunk_base = (k * _NSC + c) * _CH

        row0 = s * _ROWS_PER_TEC
        pltpu.sync_copy(cand_hbm.at[pl.ds(chunk_base + row0, _ROWS_PER_TEC)],
                        chunk_sp.at[pl.ds(row0, _ROWS_PER_TEC)])
        plsc.subcore_barrier()

        def scan_body(i, cnt):
            iv = idx_v[pl.ds(i * 16, 16)]
            rel = iv - chunk_base
            m = (rel >= 0) & (rel < _CH)
            mi = m.astype(jnp.int32)
            dest = cnt + plsc.cumsum(mi) - mi
            pos = idx_base + i * 16 + lane
            plsc.store_scatter(pos_flat, [dest], pos, mask=m)
            plsc.store_scatter(rel_flat, [dest], rel, mask=m)
            return cnt + jnp.sum(mi)

        cnt = lax.fori_loop(0, _SCAN_IT, scan_body, jnp.int32(0))

        for t in range(_BATCH // 16):
            dest = cnt + t * 16 + lane
            plsc.store_scatter(pos_flat, [dest], idx_base + t * 16 + lane)
            plsc.store_scatter(rel_flat, [dest], lane * 0 + (_CH + s))

        nb = (cnt + (_BATCH - 1)) // _BATCH

        def batch_body(j, _):
            pltpu.sync_copy(pos_flat.at[pl.ds(j * _BATCH, _BATCH)], pos_b)
            pltpu.sync_copy(rel_flat.at[pl.ds(j * _BATCH, _BATCH)], rel_b)
            pltpu.async_copy(upd_hbm.at[pos_b], rows_v, sem).wait()
            pltpu.sync_copy(rows_v, chunk_sp.at[rel_b], add=True)
            return _

        lax.fori_loop(0, nb, batch_body, jnp.int32(0))
        plsc.subcore_barrier()

        pltpu.sync_copy(chunk_sp.at[pl.ds(row0, _ROWS_PER_TEC)],
                        out_hbm.at[pl.ds(chunk_base + row0, _ROWS_PER_TEC)])
        plsc.subcore_barrier()


def kernel(candidate_rep, graph_rep, graph_sizes, put_indices, Wu, bu, Ws, bs):
    upd = _scored_update(candidate_rep, graph_rep, Wu, bu, Ws, bs)
    out = _scatter_add(candidate_rep, put_indices, upd)
    return out, graph_rep


# speedup vs baseline: 1.9188x; 1.9188x over previous
import functools

import jax
import jax.numpy as jnp
from jax import lax
from jax.experimental import pallas as pl
from jax.experimental.pallas import tpu as pltpu
from jax.experimental.pallas import tpu_sc as plsc

N = 160000
ENC = 128
D_IN = 3 * ENC


_ROWS = 640
_GRID = N // _ROWS
_WCOLS = 256


def _scored_update_body(cand_ref, graph_ref, w1_ref, w2_ref, b_ref, out_ref):
    x1 = cand_ref[...].astype(jnp.bfloat16)
    x2 = graph_ref[...].astype(jnp.bfloat16)
    y = (jnp.dot(x1, w1_ref[...], preferred_element_type=jnp.float32)
         + jnp.dot(x2, w2_ref[...], preferred_element_type=jnp.float32)
         + b_ref[...])
    u = y[:, :ENC]
    s = y[:, ENC:ENC + 1]
    out_ref[...] = u * s


def _scored_update(candidate_rep, graph_rep, Wu, bu, Ws, bs):
    w = jnp.zeros((D_IN, _WCOLS), jnp.float32)
    w = w.at[:, :ENC].set(Wu).at[:, ENC].set(Ws[:, 0])
    w = w.astype(jnp.bfloat16)
    b = jnp.zeros((1, _WCOLS), jnp.float32)
    b = b.at[0, :ENC].set(bu).at[0, ENC].set(bs[0])
    return pl.pallas_call(
        _scored_update_body,
        grid=(_GRID,),
        in_specs=[
            pl.BlockSpec((_ROWS, ENC), lambda i: (i, 0)),
            pl.BlockSpec((_ROWS, 2 * ENC), lambda i: (i, 0)),
            pl.BlockSpec((ENC, _WCOLS), lambda i: (0, 0)),
            pl.BlockSpec((2 * ENC, _WCOLS), lambda i: (0, 0)),
            pl.BlockSpec((1, _WCOLS), lambda i: (0, 0)),
        ],
        out_specs=pl.BlockSpec((_ROWS, ENC), lambda i: (i, 0)),
        out_shape=jax.ShapeDtypeStruct((N, ENC), jnp.float32),
    )(candidate_rep, graph_rep, w[:ENC], w[ENC:], b)


_NTEC = 16
_PASS = 16384
_HALF = _PASS // 2
_OWN = _HALF // _NTEC
_NPF = 9
_LWIN = N - _NPF * _PASS
_LHALF = _LWIN // 2
_LOWN = _LHALF // _NTEC
_GUARDR = 8
_PER_TEC = N // _NTEC
_SCAN_IT = _PER_TEC // 16
_BATCH = 128
_BLK = 2048
_PBUF = 5 * _BLK
_OBUF = _BLK + 3 * _BATCH
_PMASK = (1 << 18) - 1


@functools.partial(
    pl.kernel,
    out_type=jax.ShapeDtypeStruct((N, ENC), jnp.float32),
    mesh=plsc.VectorSubcoreMesh(core_axis_name="c", subcore_axis_name="s"),
    compiler_params=pltpu.CompilerParams(needs_layout_passes=False),
    scratch_types=[
        pltpu.VMEM((_PER_TEC,), jnp.int32),
        pltpu.VMEM((_PBUF,), jnp.int32),
        pltpu.VMEM((_OBUF,), jnp.int32),
        pltpu.VMEM((_BLK,), jnp.int32),
        pltpu.VMEM((256,), jnp.int32),
        pltpu.VMEM((16,), jnp.int32),
        pltpu.VMEM((16,), jnp.int32),
        pltpu.VMEM((_BATCH,), jnp.int32),
        pltpu.VMEM((_BATCH, ENC), jnp.float32),
        pltpu.VMEM((_OWN + _GUARDR, ENC), jnp.float32),
        pltpu.VMEM_SHARED((_NTEC, _PBUF), jnp.int32),
        pltpu.VMEM_SHARED((256,), jnp.int32),
        pltpu.SemaphoreType.DMA,
    ],
)
def _scatter_add(cand_hbm, idx_hbm, upd_hbm, out_hbm,
                 idx_v, prec, orec, blk_v, cnts_v, cnt_b, stage_v, pos_b,
                 rows_v, acc, mail, cnts_sp, sem):
    c = lax.axis_index("c")
    s = lax.axis_index("s")
    lane = lax.iota(jnp.int32, 16)
    idx_base = s * _PER_TEC

    pltpu.sync_copy(idx_hbm.at[pl.ds(idx_base, _PER_TEC)], idx_v)

    def append(buf, cnt, comb, m):
        pcv = plsc.all_reduce_population_count(m)
        plsc.store_compressed(stage_v.at[:], comb, mask=m)
        w = stage_v[...]
        plsc.store_scatter(buf, [cnt + lane], w, mask=lane < pcv)
        return cnt + pcv[0]

    def apply_batches(nb):
        def batch_body(jb, carry):
            for t in range(_BATCH // 16):
                w = orec[pl.ds(jb * _BATCH + t * 16, 16)]
                pos_b[pl.ds(t * 16, 16)] = w & _PMASK
            pltpu.async_copy(upd_hbm.at[pos_b], rows_v, sem).wait()

            def grp(g, carry2):
                w8 = orec[pl.ds(jb * _BATCH + g * 16, 16)]
                loc = w8 >> 18
                for j in range(16):
                    lj = loc[j]
                    for t in range(ENC // 16):
                        acc[lj, pl.ds(t * 16, 16)] = (
                            acc[lj, pl.ds(t * 16, 16)]
                            + rows_v[g * 16 + j, pl.ds(t * 16, 16)])
                return carry2

            lax.fori_loop(0, _BATCH // 16, grp, jnp.int32(0))
            return carry

        lax.fori_loop(0, nb, batch_body, jnp.int32(0))

    def run_pass(win_base, half_sz, own_sz):
        sc_base = win_base + c * half_sz
        own_lo = s * own_sz
        gbase = sc_base + own_lo

        pltpu.sync_copy(cand_hbm.at[pl.ds(gbase, own_sz)],
                        acc.at[pl.ds(0, own_sz)])

        def scan_body(i, cnt):
            iv = idx_v[pl.ds(i * 16, 16)]
            rel = iv - sc_base
            m = (rel >= 0) & (rel < half_sz)
            comb = (idx_base + i * 16 + lane) | (rel << 18)
            return append(prec, cnt, comb, m)

        cnt = lax.fori_loop(0, _SCAN_IT, scan_body, jnp.int32(0))

        cnt_b[pl.ds(0, 16)] = lane * 0 + cnt
        pltpu.sync_copy(cnt_b, cnts_sp.at[pl.ds(s * 16, 16)])
        nblk = (cnt + (_BLK - 1)) // _BLK

        def pub(b, carry):
            pltpu.sync_copy(prec.at[pl.ds(b * _BLK, _BLK)],
                            mail.at[s, pl.ds(b * _BLK, _BLK)])
            return carry

        lax.fori_loop(0, nblk, pub, jnp.int32(0))
        plsc.subcore_barrier()

        pltpu.sync_copy(cnts_sp, cnts_v)

        def con_q(q, ocnt):
            cq = cnts_v[pl.ds(q * 16, 16)][0]
            nbq = (cq + (_BLK - 1)) // _BLK

            def con_b(b, ocnt):
                pltpu.sync_copy(mail.at[q, pl.ds(b * _BLK, _BLK)], blk_v)
                remaining = cq - b * _BLK
                nvr = (jnp.minimum(remaining, _BLK) + 15) // 16

                def f_v(v, ocnt):
                    w = blk_v[pl.ds(v * 16, 16)]
                    rl = (w >> 18) - own_lo
                    vm = ((rl >= 0) & (rl < own_sz)
                          & (b * _BLK + v * 16 + lane < cq))
                    comb2 = (w & _PMASK) | (rl << 18)
                    return append(orec, ocnt, comb2, vm)

                ocnt = lax.fori_loop(0, nvr, f_v, ocnt)
                nfull = ocnt // _BATCH
                apply_batches(nfull)
                for t in range(_BATCH // 16):
                    w = orec[pl.ds(nfull * _BATCH + t * 16, 16)]
                    orec[pl.ds(t * 16, 16)] = w
                return ocnt - nfull * _BATCH

            return lax.fori_loop(0, nbq, con_b, ocnt)

        ocnt = lax.fori_loop(0, _NTEC, con_q, jnp.int32(0))

        for t in range(_BATCH // 16):
            dest = ocnt + t * 16 + lane
            padv = ((idx_base + t * 16 + lane)
                    | ((own_sz + (lane & 7)) << 18))
            plsc.store_scatter(orec, [dest], padv)
        apply_batches((ocnt + (_BATCH - 1)) // _BATCH)
        plsc.subcore_barrier()

        pltpu.sync_copy(acc.at[pl.ds(0, own_sz)],
                        out_hbm.at[pl.ds(gbase, own_sz)])
        plsc.subcore_barrier()

    def full_pass(p, carry):
        run_pass(p * _PASS, _HALF, _OWN)
        return carry

    lax.fori_loop(0, _NPF, full_pass, jnp.int32(0))
    run_pass(_NPF * _PASS, _LHALF, _LOWN)


def kernel(candidate_rep, graph_rep, graph_sizes, put_indices, Wu, bu, Ws, bs):
    upd = _scored_update(candidate_rep, graph_rep, Wu, bu, Ws, bs)
    out = _scatter_add(candidate_rep, put_indices, upd)
    return out, graph_rep


# speedup vs baseline: 2.2180x; 1.1559x over previous
import functools

import jax
import jax.numpy as jnp
from jax import lax
from jax.experimental import pallas as pl
from jax.experimental.pallas import tpu as pltpu
from jax.experimental.pallas import tpu_sc as plsc

N = 160000
ENC = 128
D_IN = 3 * ENC


_ROWS = 3200
_GRID = N // _ROWS
_WCOLS = 256


def _scored_update_body(cand_ref, graph_ref, w1_ref, w2_ref, b_ref, out_ref):
    x1 = cand_ref[...].astype(jnp.bfloat16)
    x2 = graph_ref[...].astype(jnp.bfloat16)
    y = (jnp.dot(x1, w1_ref[...], preferred_element_type=jnp.float32)
         + jnp.dot(x2, w2_ref[...], preferred_element_type=jnp.float32)
         + b_ref[...])
    u = y[:, :ENC]
    s = y[:, ENC:ENC + 1]
    out_ref[...] = u * s


def _scored_update(candidate_rep, graph_rep, Wu, bu, Ws, bs):
    w = jnp.zeros((D_IN, _WCOLS), jnp.float32)
    w = w.at[:, :ENC].set(Wu).at[:, ENC].set(Ws[:, 0])
    w = w.astype(jnp.bfloat16)
    b = jnp.zeros((1, _WCOLS), jnp.float32)
    b = b.at[0, :ENC].set(bu).at[0, ENC].set(bs[0])
    return pl.pallas_call(
        _scored_update_body,
        grid=(_GRID,),
        in_specs=[
            pl.BlockSpec((_ROWS, ENC), lambda i: (i, 0)),
            pl.BlockSpec((_ROWS, 2 * ENC), lambda i: (i, 0)),
            pl.BlockSpec((ENC, _WCOLS), lambda i: (0, 0)),
            pl.BlockSpec((2 * ENC, _WCOLS), lambda i: (0, 0)),
            pl.BlockSpec((1, _WCOLS), lambda i: (0, 0)),
        ],
        out_specs=pl.BlockSpec((_ROWS, ENC), lambda i: (i, 0)),
        out_shape=jax.ShapeDtypeStruct((N, ENC), jnp.float32),
    )(candidate_rep, graph_rep, w[:ENC], w[ENC:], b)


_NTEC = 16
_PASS = 16384
_HALF = _PASS // 2
_OWN = _HALF // _NTEC
_NPF = 9
_LWIN = N - _NPF * _PASS
_LHALF = _LWIN // 2
_LOWN = _LHALF // _NTEC
_GUARDR = 8
_PER_TEC = N // _NTEC
_SCAN_IT = _PER_TEC // 16
_BATCH = 128
_BLK = 2048
_PBUF = 5 * _BLK
_OBUF = _BLK + 3 * _BATCH
_PMASK = (1 << 18) - 1


@functools.partial(
    pl.kernel,
    out_type=jax.ShapeDtypeStruct((N, ENC), jnp.float32),
    mesh=plsc.VectorSubcoreMesh(core_axis_name="c", subcore_axis_name="s"),
    compiler_params=pltpu.CompilerParams(needs_layout_passes=False),
    scratch_types=[
        pltpu.VMEM((_PER_TEC,), jnp.int32),
        pltpu.VMEM((_PBUF,), jnp.int32),
        pltpu.VMEM((_OBUF,), jnp.int32),
        pltpu.VMEM((_BLK,), jnp.int32),
        pltpu.VMEM((256,), jnp.int32),
        pltpu.VMEM((16,), jnp.int32),
        pltpu.VMEM((16,), jnp.int32),
        pltpu.VMEM((_BATCH,), jnp.int32),
        pltpu.VMEM((_BATCH, ENC), jnp.float32),
        pltpu.VMEM((_OWN + _GUARDR, ENC), jnp.float32),
        pltpu.VMEM_SHARED((_NTEC, _PBUF), jnp.int32),
        pltpu.VMEM_SHARED((256,), jnp.int32),
        pltpu.SemaphoreType.DMA,
    ],
)
def _scatter_add(cand_hbm, idx_hbm, upd_hbm, out_hbm,
                 idx_v, prec, orec, blk_v, cnts_v, cnt_b, stage_v, pos_b,
                 rows_v, acc, mail, cnts_sp, sem):
    c = lax.axis_index("c")
    s = lax.axis_index("s")
    lane = lax.iota(jnp.int32, 16)
    idx_base = s * _PER_TEC

    pltpu.sync_copy(idx_hbm.at[pl.ds(idx_base, _PER_TEC)], idx_v)

    def append(buf, cnt, comb, m):
        pcv = plsc.all_reduce_population_count(m)
        plsc.store_compressed(stage_v.at[:], comb, mask=m)
        w = stage_v[...]
        plsc.store_scatter(buf, [cnt + lane], w, mask=lane < pcv)
        return cnt + pcv[0]

    def apply_batches(nb):
        def batch_body(jb, carry):
            for t in range(_BATCH // 16):
                w = orec[pl.ds(jb * _BATCH + t * 16, 16)]
                pos_b[pl.ds(t * 16, 16)] = w & _PMASK
            pltpu.async_copy(upd_hbm.at[pos_b], rows_v, sem).wait()

            def grp(g, carry2):
                w8 = orec[pl.ds(jb * _BATCH + g * 16, 16)]
                loc = w8 >> 18
                for j in range(16):
                    lj = loc[j]
                    for t in range(ENC // 16):
                        acc[lj, pl.ds(t * 16, 16)] = (
                            acc[lj, pl.ds(t * 16, 16)]
                            + rows_v[g * 16 + j, pl.ds(t * 16, 16)])
                return carry2

            lax.fori_loop(0, _BATCH // 16, grp, jnp.int32(0))
            return carry

        lax.fori_loop(0, nb, batch_body, jnp.int32(0))

    def run_pass(win_base, half_sz, own_sz):
        sc_base = win_base + c * half_sz
        own_lo = s * own_sz
        gbase = sc_base + own_lo

        pltpu.sync_copy(cand_hbm.at[pl.ds(gbase, own_sz)],
                        acc.at[pl.ds(0, own_sz)])

        def scan_body(i, cnt):
            iv = idx_v[pl.ds(i * 16, 16)]
            rel = iv - sc_base
            m = (rel >= 0) & (rel < half_sz)
            comb = (idx_base + i * 16 + lane) | (rel << 18)
            return append(prec, cnt, comb, m)

        cnt = lax.fori_loop(0, _SCAN_IT, scan_body, jnp.int32(0))

        cnt_b[pl.ds(0, 16)] = lane * 0 + cnt
        pltpu.sync_copy(cnt_b, cnts_sp.at[pl.ds(s * 16, 16)])
        nblk = (cnt + (_BLK - 1)) // _BLK

        def pub(b, carry):
            pltpu.sync_copy(prec.at[pl.ds(b * _BLK, _BLK)],
                            mail.at[s, pl.ds(b * _BLK, _BLK)])
            return carry

        lax.fori_loop(0, nblk, pub, jnp.int32(0))
        plsc.subcore_barrier()

        pltpu.sync_copy(cnts_sp, cnts_v)

        def con_q(q, ocnt):
            cq = cnts_v[pl.ds(q * 16, 16)][0]
            nbq = (cq + (_BLK - 1)) // _BLK

            def con_b(b, ocnt):
                pltpu.sync_copy(mail.at[q, pl.ds(b * _BLK, _BLK)], blk_v)
                remaining = cq - b * _BLK
                nvr = (jnp.minimum(remaining, _BLK) + 15) // 16

                def f_v(v, ocnt):
                    w = blk_v[pl.ds(v * 16, 16)]
                    rl = (w >> 18) - own_lo
                    vm = ((rl >= 0) & (rl < own_sz)
                          & (b * _BLK + v * 16 + lane < cq))
                    comb2 = (w & _PMASK) | (rl << 18)
                    return append(orec, ocnt, comb2, vm)

                ocnt = lax.fori_loop(0, nvr, f_v, ocnt)
                nfull = ocnt // _BATCH
                apply_batches(nfull)
                for t in range(_BATCH // 16):
                    w = orec[pl.ds(nfull * _BATCH + t * 16, 16)]
                    orec[pl.ds(t * 16, 16)] = w
                return ocnt - nfull * _BATCH

            return lax.fori_loop(0, nbq, con_b, ocnt)

        ocnt = lax.fori_loop(0, _NTEC, con_q, jnp.int32(0))

        for t in range(_BATCH // 16):
            dest = ocnt + t * 16 + lane
            padv = ((idx_base + t * 16 + lane)
                    | ((own_sz + (lane & 7)) << 18))
            plsc.store_scatter(orec, [dest], padv)
        apply_batches((ocnt + (_BATCH - 1)) // _BATCH)
        plsc.subcore_barrier()

        pltpu.sync_copy(acc.at[pl.ds(0, own_sz)],
                        out_hbm.at[pl.ds(gbase, own_sz)])
        plsc.subcore_barrier()

    def full_pass(p, carry):
        run_pass(p * _PASS, _HALF, _OWN)
        return carry

    lax.fori_loop(0, _NPF, full_pass, jnp.int32(0))
    run_pass(_NPF * _PASS, _LHALF, _LOWN)


def kernel(candidate_rep, graph_rep, graph_sizes, put_indices, Wu, bu, Ws, bs):
    upd = _scored_update(candidate_rep, graph_rep, Wu, bu, Ws, bs)
    out = _scatter_add(candidate_rep, put_indices, upd)
    return out, graph_rep


# speedup vs baseline: 2.2486x; 1.0138x over previous
import functools

import jax
import jax.numpy as jnp
from jax import lax
from jax.experimental import pallas as pl
from jax.experimental.pallas import tpu as pltpu
from jax.experimental.pallas import tpu_sc as plsc

N = 160000
ENC = 128
D_IN = 3 * ENC


_ROWS = 8000
_GRID = N // _ROWS
_WCOLS = 256


def _scored_update_body(cand_ref, graph_ref, w1_ref, w2_ref, b_ref, out_ref):
    x1 = cand_ref[...].astype(jnp.bfloat16)
    x2 = graph_ref[...].astype(jnp.bfloat16)
    y = (jnp.dot(x1, w1_ref[...], preferred_element_type=jnp.float32)
         + jnp.dot(x2, w2_ref[...], preferred_element_type=jnp.float32)
         + b_ref[...])
    u = y[:, :ENC]
    s = y[:, ENC:ENC + 1]
    out_ref[...] = u * s


def _scored_update(candidate_rep, graph_rep, Wu, bu, Ws, bs):
    w = jnp.zeros((D_IN, _WCOLS), jnp.float32)
    w = w.at[:, :ENC].set(Wu).at[:, ENC].set(Ws[:, 0])
    w = w.astype(jnp.bfloat16)
    b = jnp.zeros((1, _WCOLS), jnp.float32)
    b = b.at[0, :ENC].set(bu).at[0, ENC].set(bs[0])
    return pl.pallas_call(
        _scored_update_body,
        grid=(_GRID,),
        in_specs=[
            pl.BlockSpec((_ROWS, ENC), lambda i: (i, 0)),
            pl.BlockSpec((_ROWS, 2 * ENC), lambda i: (i, 0)),
            pl.BlockSpec((ENC, _WCOLS), lambda i: (0, 0)),
            pl.BlockSpec((2 * ENC, _WCOLS), lambda i: (0, 0)),
            pl.BlockSpec((1, _WCOLS), lambda i: (0, 0)),
        ],
        out_specs=pl.BlockSpec((_ROWS, ENC), lambda i: (i, 0)),
        out_shape=jax.ShapeDtypeStruct((N, ENC), jnp.float32),
    )(candidate_rep, graph_rep, w[:ENC], w[ENC:], b)


_NTEC = 16
_PASS = 16384
_HALF = _PASS // 2
_OWN = _HALF // _NTEC
_NPF = 9
_LWIN = N - _NPF * _PASS
_LHALF = _LWIN // 2
_LOWN = _LHALF // _NTEC
_GUARDR = 8
_PER_TEC = N // _NTEC
_SCAN_IT = _PER_TEC // 16
_BATCH = 128
_BLK = 2048
_PBUF = 5 * _BLK
_OBUF = _BLK + 3 * _BATCH
_PMASK = (1 << 18) - 1


@functools.partial(
    pl.kernel,
    out_type=jax.ShapeDtypeStruct((N, ENC), jnp.float32),
    mesh=plsc.VectorSubcoreMesh(core_axis_name="c", subcore_axis_name="s"),
    compiler_params=pltpu.CompilerParams(needs_layout_passes=False),
    scratch_types=[
        pltpu.VMEM((_PER_TEC,), jnp.int32),
        pltpu.VMEM((_PBUF,), jnp.int32),
        pltpu.VMEM((_OBUF,), jnp.int32),
        pltpu.VMEM((_BLK,), jnp.int32),
        pltpu.VMEM((256,), jnp.int32),
        pltpu.VMEM((16,), jnp.int32),
        pltpu.VMEM((16,), jnp.int32),
        pltpu.VMEM((_BATCH,), jnp.int32),
        pltpu.VMEM((_BATCH, ENC), jnp.float32),
        pltpu.VMEM((_OWN + _GUARDR, ENC), jnp.float32),
        pltpu.VMEM_SHARED((_NTEC, _PBUF), jnp.int32),
        pltpu.VMEM_SHARED((256,), jnp.int32),
        pltpu.SemaphoreType.DMA,
    ],
)
def _scatter_add(cand_hbm, idx_hbm, upd_hbm, out_hbm,
                 idx_v, prec, orec, blk_v, cnts_v, cnt_b, stage_v, pos_b,
                 rows_v, acc, mail, cnts_sp, sem):
    c = lax.axis_index("c")
    s = lax.axis_index("s")
    lane = lax.iota(jnp.int32, 16)
    idx_base = s * _PER_TEC

    pltpu.sync_copy(idx_hbm.at[pl.ds(idx_base, _PER_TEC)], idx_v)

    def append(buf, cnt, comb, m):
        pcv = plsc.all_reduce_population_count(m)
        plsc.store_compressed(stage_v.at[:], comb, mask=m)
        w = stage_v[...]
        plsc.store_scatter(buf, [cnt + lane], w, mask=lane < pcv)
        return cnt + pcv[0]

    def apply_batches(nb):
        def batch_body(jb, carry):
            for t in range(_BATCH // 16):
                w = orec[pl.ds(jb * _BATCH + t * 16, 16)]
                pos_b[pl.ds(t * 16, 16)] = w & _PMASK
            pltpu.async_copy(upd_hbm.at[pos_b], rows_v, sem).wait()

            def grp(g, carry2):
                w8 = orec[pl.ds(jb * _BATCH + g * 16, 16)]
                loc = w8 >> 18
                for j in range(16):
                    lj = loc[j]
                    for t in range(ENC // 16):
                        acc[lj, pl.ds(t * 16, 16)] = (
                            acc[lj, pl.ds(t * 16, 16)]
                            + rows_v[g * 16 + j, pl.ds(t * 16, 16)])
                return carry2

            lax.fori_loop(0, _BATCH // 16, grp, jnp.int32(0))
            return carry

        lax.fori_loop(0, nb, batch_body, jnp.int32(0))

    def run_pass(win_base, half_sz, own_sz):
        sc_base = win_base + c * half_sz
        own_lo = s * own_sz
        gbase = sc_base + own_lo

        pltpu.sync_copy(cand_hbm.at[pl.ds(gbase, own_sz)],
                        acc.at[pl.ds(0, own_sz)])

        def scan_body(i, cnt):
            iv = idx_v[pl.ds(i * 16, 16)]
            rel = iv - sc_base
            m = (rel >= 0) & (rel < half_sz)
            comb = (idx_base + i * 16 + lane) | (rel << 18)
            return append(prec, cnt, comb, m)

        cnt = lax.fori_loop(0, _SCAN_IT, scan_body, jnp.int32(0))

        cnt_b[pl.ds(0, 16)] = lane * 0 + cnt
        pltpu.sync_copy(cnt_b, cnts_sp.at[pl.ds(s * 16, 16)])
        nblk = (cnt + (_BLK - 1)) // _BLK

        def pub(b, carry):
            pltpu.sync_copy(prec.at[pl.ds(b * _BLK, _BLK)],
                            mail.at[s, pl.ds(b * _BLK, _BLK)])
            return carry

        lax.fori_loop(0, nblk, pub, jnp.int32(0))
        plsc.subcore_barrier()

        pltpu.sync_copy(cnts_sp, cnts_v)

        def con_q(q, ocnt):
            cq = cnts_v[pl.ds(q * 16, 16)][0]
            nbq = (cq + (_BLK - 1)) // _BLK

            def con_b(b, ocnt):
                pltpu.sync_copy(mail.at[q, pl.ds(b * _BLK, _BLK)], blk_v)
                remaining = cq - b * _BLK
                nvr = (jnp.minimum(remaining, _BLK) + 15) // 16

                def f_v(v, ocnt):
                    w = blk_v[pl.ds(v * 16, 16)]
                    rl = (w >> 18) - own_lo
                    vm = ((rl >= 0) & (rl < own_sz)
                          & (b * _BLK + v * 16 + lane < cq))
                    comb2 = (w & _PMASK) | (rl << 18)
                    return append(orec, ocnt, comb2, vm)

                ocnt = lax.fori_loop(0, nvr, f_v, ocnt)
                nfull = ocnt // _BATCH
                apply_batches(nfull)
                for t in range(_BATCH // 16):
                    w = orec[pl.ds(nfull * _BATCH + t * 16, 16)]
                    orec[pl.ds(t * 16, 16)] = w
                return ocnt - nfull * _BATCH

            return lax.fori_loop(0, nbq, con_b, ocnt)

        ocnt = lax.fori_loop(0, _NTEC, con_q, jnp.int32(0))

        for t in range(_BATCH // 16):
            dest = ocnt + t * 16 + lane
            padv = ((idx_base + t * 16 + lane)
                    | ((own_sz + (lane & 7)) << 18))
            plsc.store_scatter(orec, [dest], padv)
        apply_batches((ocnt + (_BATCH - 1)) // _BATCH)
        plsc.subcore_barrier()

        pltpu.sync_copy(acc.at[pl.ds(0, own_sz)],
                        out_hbm.at[pl.ds(gbase, own_sz)])
        plsc.subcore_barrier()

    def full_pass(p, carry):
        run_pass(p * _PASS, _HALF, _OWN)
        return carry

    lax.fori_loop(0, _NPF, full_pass, jnp.int32(0))
    run_pass(_NPF * _PASS, _LHALF, _LOWN)


def kernel(candidate_rep, graph_rep, graph_sizes, put_indices, Wu, bu, Ws, bs):
    upd = _scored_update(candidate_rep, graph_rep, Wu, bu, Ws, bs)
    out = _scatter_add(candidate_rep, put_indices, upd)
    return out, graph_rep


# speedup vs baseline: 2.3351x; 1.0385x over previous
import functools

import jax
import jax.numpy as jnp
from jax import lax
from jax.experimental import pallas as pl
from jax.experimental.pallas import tpu as pltpu
from jax.experimental.pallas import tpu_sc as plsc

N = 160000
ENC = 128
D_IN = 3 * ENC


_ROWS = 8000
_GRID = N // _ROWS
_WCOLS = 256


def _scored_update_body(cand_ref, graph_ref, w1_ref, w2_ref, b_ref, out_ref):
    x1 = cand_ref[...].astype(jnp.bfloat16)
    x2 = graph_ref[...].astype(jnp.bfloat16)
    y = (jnp.dot(x1, w1_ref[...], preferred_element_type=jnp.float32)
         + jnp.dot(x2, w2_ref[...], preferred_element_type=jnp.float32)
         + b_ref[...])
    u = y[:, :ENC]
    s = y[:, ENC:ENC + 1]
    out_ref[...] = u * s


def _scored_update(candidate_rep, graph_rep, Wu, bu, Ws, bs):
    w = jnp.zeros((D_IN, _WCOLS), jnp.float32)
    w = w.at[:, :ENC].set(Wu).at[:, ENC].set(Ws[:, 0])
    w = w.astype(jnp.bfloat16)
    b = jnp.zeros((1, _WCOLS), jnp.float32)
    b = b.at[0, :ENC].set(bu).at[0, ENC].set(bs[0])
    return pl.pallas_call(
        _scored_update_body,
        grid=(_GRID,),
        in_specs=[
            pl.BlockSpec((_ROWS, ENC), lambda i: (i, 0)),
            pl.BlockSpec((_ROWS, 2 * ENC), lambda i: (i, 0)),
            pl.BlockSpec((ENC, _WCOLS), lambda i: (0, 0)),
            pl.BlockSpec((2 * ENC, _WCOLS), lambda i: (0, 0)),
            pl.BlockSpec((1, _WCOLS), lambda i: (0, 0)),
        ],
        out_specs=pl.BlockSpec((_ROWS, ENC), lambda i: (i, 0)),
        out_shape=jax.ShapeDtypeStruct((N, ENC), jnp.float32),
    )(candidate_rep, graph_rep, w[:ENC], w[ENC:], b)


_NTEC = 16
_PASS = 16384
_HALF = _PASS // 2
_OWN = _HALF // _NTEC
_NPF = 9
_LWIN = N - _NPF * _PASS
_LHALF = _LWIN // 2
_LOWN = _LHALF // _NTEC
_GUARDR = 8
_PER_TEC = N // _NTEC
_SCAN_IT = _PER_TEC // 16
_BATCH = 128
_BLK = 2048
_PBUF = 5 * _BLK
_OBUF = _BLK + 3 * _BATCH
_PMASK = (1 << 18) - 1


@functools.partial(
    pl.kernel,
    out_type=jax.ShapeDtypeStruct((N, ENC), jnp.float32),
    mesh=plsc.VectorSubcoreMesh(core_axis_name="c", subcore_axis_name="s"),
    compiler_params=pltpu.CompilerParams(needs_layout_passes=False),
    scratch_types=[
        pltpu.VMEM((_PER_TEC,), jnp.int32),
        pltpu.VMEM((_PBUF,), jnp.int32),
        pltpu.VMEM((_OBUF,), jnp.int32),
        pltpu.VMEM((_BLK,), jnp.int32),
        pltpu.VMEM((256,), jnp.int32),
        pltpu.VMEM((16,), jnp.int32),
        pltpu.VMEM((32,), jnp.int32),
        pltpu.VMEM((64,), jnp.int32),
        pltpu.VMEM((64,), jnp.int32),
        pltpu.VMEM((_BATCH, ENC), jnp.float32),
        pltpu.VMEM((_OWN + _GUARDR, ENC), jnp.float32),
        pltpu.VMEM_SHARED((_NTEC, _PBUF), jnp.int32),
        pltpu.VMEM_SHARED((256,), jnp.int32),
        pltpu.SemaphoreType.DMA,
    ],
)
def _scatter_add(cand_hbm, idx_hbm, upd_hbm, out_hbm,
                 idx_v, prec, orec, blk_v, cnts_v, cnt_b, stage_v, pos_b0,
                 pos_b1, rows_v, acc, mail, cnts_sp, sem):
    c = lax.axis_index("c")
    s = lax.axis_index("s")
    lane = lax.iota(jnp.int32, 16)
    idx_base = s * _PER_TEC

    pltpu.sync_copy(idx_hbm.at[pl.ds(idx_base, _PER_TEC)], idx_v)

    def append(buf, cnt, comb, m):
        pcv = plsc.all_reduce_population_count(m)
        plsc.store_compressed(stage_v.at[pl.ds(0, 16)], comb, mask=m)
        w = stage_v[pl.ds(0, 16)]
        plsc.store_scatter(buf, [cnt + lane], w, mask=lane < pcv)
        return cnt + pcv[0]

    def decode(hb, pos_b):
        for t in range(4):
            w = orec[pl.ds(hb * 64 + t * 16, 16)]
            pos_b[pl.ds(t * 16, 16)] = w & _PMASK

    def start(pos_b, half):
        return pltpu.async_copy(upd_hbm.at[pos_b],
                                rows_v.at[pl.ds(half * 64, 64)], sem)

    def drain(half):
        pltpu.make_async_copy(upd_hbm.at[pos_b0],
                              rows_v.at[pl.ds(half * 64, 64)], sem).wait()

    def apply64(hb, half):
        def grp(g, carry2):
            w8 = orec[pl.ds(hb * 64 + g * 16, 16)]
            loc = w8 >> 18
            for j in range(16):
                lj = loc[j]
                for t in range(ENC // 16):
                    acc[lj, pl.ds(t * 16, 16)] = (
                        acc[lj, pl.ds(t * 16, 16)]
                        + rows_v[half * 64 + g * 16 + j, pl.ds(t * 16, 16)])
            return carry2

        lax.fori_loop(0, 4, grp, jnp.int32(0))

    def apply_batches(npair):
        @pl.when(npair > 0)
        def _pipe():
            decode(0, pos_b0)
            start(pos_b0, 0)

            def pair(pb, carry):
                hb0 = 2 * pb
                decode(hb0 + 1, pos_b1)
                drain(0)
                start(pos_b1, 1)
                apply64(hb0, 0)

                @pl.when(pb + 1 < npair)
                def _pf():
                    decode(hb0 + 2, pos_b0)

                drain(1)

                @pl.when(pb + 1 < npair)
                def _st():
                    start(pos_b0, 0)

                apply64(hb0 + 1, 1)
                return carry

            lax.fori_loop(0, npair, pair, jnp.int32(0))

    def run_pass(win_base, half_sz, own_sz):
        sc_base = win_base + c * half_sz
        own_lo = s * own_sz
        gbase = sc_base + own_lo

        pltpu.sync_copy(cand_hbm.at[pl.ds(gbase, own_sz)],
                        acc.at[pl.ds(0, own_sz)])

        def scan_body(i, cnt):
            iv0 = idx_v[pl.ds(i * 32, 16)]
            iv1 = idx_v[pl.ds(i * 32 + 16, 16)]
            rel0 = iv0 - sc_base
            rel1 = iv1 - sc_base
            m0 = (rel0 >= 0) & (rel0 < half_sz)
            m1 = (rel1 >= 0) & (rel1 < half_sz)
            comb0 = (idx_base + i * 32 + lane) | (rel0 << 18)
            comb1 = (idx_base + i * 32 + 16 + lane) | (rel1 << 18)
            pcv0 = plsc.all_reduce_population_count(m0)
            pcv1 = plsc.all_reduce_population_count(m1)
            plsc.store_compressed(stage_v.at[pl.ds(0, 16)], comb0, mask=m0)
            plsc.store_compressed(stage_v.at[pl.ds(16, 16)], comb1, mask=m1)
            w0 = stage_v[pl.ds(0, 16)]
            w1 = stage_v[pl.ds(16, 16)]
            cnt1 = cnt + pcv0[0]
            plsc.store_scatter(prec, [cnt + lane], w0, mask=lane < pcv0)
            plsc.store_scatter(prec, [cnt1 + lane], w1, mask=lane < pcv1)
            return cnt1 + pcv1[0]

        cnt = lax.fori_loop(0, _SCAN_IT // 2, scan_body, jnp.int32(0))
        ivt = idx_v[pl.ds(_PER_TEC - 16, 16)]
        relt = ivt - sc_base
        mt = (relt >= 0) & (relt < half_sz)
        cnt = append(prec, cnt,
                     (idx_base + _PER_TEC - 16 + lane) | (relt << 18), mt)

        cnt_b[pl.ds(0, 16)] = lane * 0 + cnt
        pltpu.sync_copy(cnt_b, cnts_sp.at[pl.ds(s * 16, 16)])
        nblk = (cnt + (_BLK - 1)) // _BLK

        def pub(b, carry):
            pltpu.sync_copy(prec.at[pl.ds(b * _BLK, _BLK)],
                            mail.at[s, pl.ds(b * _BLK, _BLK)])
            return carry

        lax.fori_loop(0, nblk, pub, jnp.int32(0))
        plsc.subcore_barrier()

        pltpu.sync_copy(cnts_sp, cnts_v)

        def con_q(q, ocnt):
            cq = cnts_v[pl.ds(q * 16, 16)][0]
            nbq = (cq + (_BLK - 1)) // _BLK

            def con_b(b, ocnt):
                pltpu.sync_copy(mail.at[q, pl.ds(b * _BLK, _BLK)], blk_v)
                remaining = cq - b * _BLK
                nvr = (jnp.minimum(remaining, _BLK) + 15) // 16

                def f_v(v, ocnt):
                    w = blk_v[pl.ds(v * 16, 16)]
                    rl = (w >> 18) - own_lo
                    vm = ((rl >= 0) & (rl < own_sz)
                          & (b * _BLK + v * 16 + lane < cq))
                    comb2 = (w & _PMASK) | (rl << 18)
                    return append(orec, ocnt, comb2, vm)

                ocnt = lax.fori_loop(0, nvr, f_v, ocnt)
                nfull = ocnt // _BATCH
                apply_batches(nfull)
                for t in range(_BATCH // 16):
                    w = orec[pl.ds(nfull * _BATCH + t * 16, 16)]
                    orec[pl.ds(t * 16, 16)] = w
                return ocnt - nfull * _BATCH

            return lax.fori_loop(0, nbq, con_b, ocnt)

        ocnt = lax.fori_loop(0, _NTEC, con_q, jnp.int32(0))

        for t in range(_BATCH // 16):
            dest = ocnt + t * 16 + lane
            padv = ((idx_base + t * 16 + lane)
                    | ((own_sz + (lane & 7)) << 18))
            plsc.store_scatter(orec, [dest], padv)
        apply_batches((ocnt + (_BATCH - 1)) // _BATCH)
        plsc.subcore_barrier()

        pltpu.sync_copy(acc.at[pl.ds(0, own_sz)],
                        out_hbm.at[pl.ds(gbase, own_sz)])
        plsc.subcore_barrier()

    def full_pass(p, carry):
        run_pass(p * _PASS, _HALF, _OWN)
        return carry

    lax.fori_loop(0, _NPF, full_pass, jnp.int32(0))
    run_pass(_NPF * _PASS, _LHALF, _LOWN)


def kernel(candidate_rep, graph_rep, graph_sizes, put_indices, Wu, bu, Ws, bs):
    upd = _scored_update(candidate_rep, graph_rep, Wu, bu, Ws, bs)
    out = _scatter_add(candidate_rep, put_indices, upd)
    return out, graph_rep


# speedup vs baseline: 2.4559x; 1.0517x over previous
import functools

import jax
import jax.numpy as jnp
from jax import lax
from jax.experimental import pallas as pl
from jax.experimental.pallas import tpu as pltpu
from jax.experimental.pallas import tpu_sc as plsc

N = 160000
ENC = 128
D_IN = 3 * ENC


_ROWS = 8000
_GRID = N // _ROWS
_WCOLS = 256


def _scored_update_body(cand_ref, graph_ref, w1_ref, w2_ref, b_ref, out_ref):
    x1 = cand_ref[...].astype(jnp.bfloat16)
    x2 = graph_ref[...].astype(jnp.bfloat16)
    y = (jnp.dot(x1, w1_ref[...], preferred_element_type=jnp.float32)
         + jnp.dot(x2, w2_ref[...], preferred_element_type=jnp.float32)
         + b_ref[...])
    u = y[:, :ENC]
    s = y[:, ENC:ENC + 1]
    out_ref[...] = u * s


def _scored_update(candidate_rep, graph_rep, Wu, bu, Ws, bs):
    w = jnp.zeros((D_IN, _WCOLS), jnp.float32)
    w = w.at[:, :ENC].set(Wu).at[:, ENC].set(Ws[:, 0])
    w = w.astype(jnp.bfloat16)
    b = jnp.zeros((1, _WCOLS), jnp.float32)
    b = b.at[0, :ENC].set(bu).at[0, ENC].set(bs[0])
    return pl.pallas_call(
        _scored_update_body,
        grid=(_GRID,),
        in_specs=[
            pl.BlockSpec((_ROWS, ENC), lambda i: (i, 0)),
            pl.BlockSpec((_ROWS, 2 * ENC), lambda i: (i, 0)),
            pl.BlockSpec((ENC, _WCOLS), lambda i: (0, 0)),
            pl.BlockSpec((2 * ENC, _WCOLS), lambda i: (0, 0)),
            pl.BlockSpec((1, _WCOLS), lambda i: (0, 0)),
        ],
        out_specs=pl.BlockSpec((_ROWS, ENC), lambda i: (i, 0)),
        out_shape=jax.ShapeDtypeStruct((N, ENC), jnp.float32),
    )(candidate_rep, graph_rep, w[:ENC], w[ENC:], b)


_NTEC = 16
_PASS = 16384
_HALF = _PASS // 2
_OWN = _HALF // _NTEC
_NPF = 9
_LWIN = N - _NPF * _PASS
_LHALF = _LWIN // 2
_LOWN = _LHALF // _NTEC
_GUARDR = 8
_PER_TEC = N // _NTEC
_SCAN_IT = _PER_TEC // 16
_BATCH = 128
_BLK = 2048
_PBUF = 5 * _BLK
_OBUF = _BLK + 3 * _BATCH
_PMASK = (1 << 18) - 1


@functools.partial(
    pl.kernel,
    out_type=jax.ShapeDtypeStruct((N, ENC), jnp.float32),
    mesh=plsc.VectorSubcoreMesh(core_axis_name="c", subcore_axis_name="s"),
    compiler_params=pltpu.CompilerParams(needs_layout_passes=False),
    scratch_types=[
        pltpu.VMEM((_PER_TEC,), jnp.int32),
        pltpu.VMEM((_PBUF,), jnp.int32),
        pltpu.VMEM((_OBUF,), jnp.int32),
        pltpu.VMEM((_BLK,), jnp.int32),
        pltpu.VMEM((256,), jnp.int32),
        pltpu.VMEM((16,), jnp.int32),
        pltpu.VMEM((32,), jnp.int32),
        pltpu.VMEM((64,), jnp.int32),
        pltpu.VMEM((64,), jnp.int32),
        pltpu.VMEM((_BATCH, ENC), jnp.float32),
        pltpu.VMEM((_OWN + _GUARDR, ENC), jnp.float32),
        pltpu.VMEM_SHARED((_NTEC, _PBUF), jnp.int32),
        pltpu.VMEM_SHARED((256,), jnp.int32),
        pltpu.SemaphoreType.DMA,
        pltpu.SemaphoreType.DMA,
    ],
)
def _scatter_add(cand_hbm, idx_hbm, upd_hbm, out_hbm,
                 idx_v, prec, orec, blk_v, cnts_v, cnt_b, stage_v, pos_b0,
                 pos_b1, rows_v, acc, mail, cnts_sp, sem, sem2):
    c = lax.axis_index("c")
    s = lax.axis_index("s")
    lane = lax.iota(jnp.int32, 16)
    idx_base = s * _PER_TEC

    pltpu.sync_copy(idx_hbm.at[pl.ds(idx_base, _PER_TEC)], idx_v)

    def append(buf, cnt, comb, m):
        pcv = plsc.all_reduce_population_count(m)
        plsc.store_compressed(stage_v.at[pl.ds(0, 16)], comb, mask=m)
        w = stage_v[pl.ds(0, 16)]
        plsc.store_scatter(buf, [cnt + lane], w, mask=lane < pcv)
        return cnt + pcv[0]

    def decode(hb, pos_b):
        for t in range(4):
            w = orec[pl.ds(hb * 64 + t * 16, 16)]
            pos_b[pl.ds(t * 16, 16)] = w & _PMASK

    def start(pos_b, half):
        return pltpu.async_copy(upd_hbm.at[pos_b],
                                rows_v.at[pl.ds(half * 64, 64)], sem)

    def drain(half):
        pltpu.make_async_copy(upd_hbm.at[pos_b0],
                              rows_v.at[pl.ds(half * 64, 64)], sem).wait()

    def apply64(hb, half):
        def grp(g, carry2):
            w8 = orec[pl.ds(hb * 64 + g * 16, 16)]
            loc = w8 >> 18
            for j in range(16):
                lj = loc[j]
                for t in range(ENC // 16):
                    acc[lj, pl.ds(t * 16, 16)] = (
                        acc[lj, pl.ds(t * 16, 16)]
                        + rows_v[half * 64 + g * 16 + j, pl.ds(t * 16, 16)])
            return carry2

        lax.fori_loop(0, 4, grp, jnp.int32(0))

    def apply_batches(npair):
        @pl.when(npair > 0)
        def _pipe():
            decode(0, pos_b0)
            start(pos_b0, 0)

            def pair(pb, carry):
                hb0 = 2 * pb
                decode(hb0 + 1, pos_b1)
                drain(0)
                start(pos_b1, 1)
                apply64(hb0, 0)

                @pl.when(pb + 1 < npair)
                def _pf():
                    decode(hb0 + 2, pos_b0)

                drain(1)

                @pl.when(pb + 1 < npair)
                def _st():
                    start(pos_b0, 0)

                apply64(hb0 + 1, 1)
                return carry

            lax.fori_loop(0, npair, pair, jnp.int32(0))

    def run_pass(win_base, half_sz, own_sz):
        sc_base = win_base + c * half_sz
        own_lo = s * own_sz
        gbase = sc_base + own_lo

        pltpu.async_copy(cand_hbm.at[pl.ds(gbase, own_sz)],
                         acc.at[pl.ds(0, own_sz)], sem2)

        def scan_body(i, cnt):
            iv0 = idx_v[pl.ds(i * 32, 16)]
            iv1 = idx_v[pl.ds(i * 32 + 16, 16)]
            rel0 = iv0 - sc_base
            rel1 = iv1 - sc_base
            m0 = (rel0 >= 0) & (rel0 < half_sz)
            m1 = (rel1 >= 0) & (rel1 < half_sz)
            comb0 = (idx_base + i * 32 + lane) | (rel0 << 18)
            comb1 = (idx_base + i * 32 + 16 + lane) | (rel1 << 18)
            pcv0 = plsc.all_reduce_population_count(m0)
            pcv1 = plsc.all_reduce_population_count(m1)
            plsc.store_compressed(stage_v.at[pl.ds(0, 16)], comb0, mask=m0)
            plsc.store_compressed(stage_v.at[pl.ds(16, 16)], comb1, mask=m1)
            w0 = stage_v[pl.ds(0, 16)]
            w1 = stage_v[pl.ds(16, 16)]
            cnt1 = cnt + pcv0[0]
            plsc.store_scatter(prec, [cnt + lane], w0, mask=lane < pcv0)
            plsc.store_scatter(prec, [cnt1 + lane], w1, mask=lane < pcv1)
            return cnt1 + pcv1[0]

        cnt = lax.fori_loop(0, _SCAN_IT // 2, scan_body, jnp.int32(0))
        ivt = idx_v[pl.ds(_PER_TEC - 16, 16)]
        relt = ivt - sc_base
        mt = (relt >= 0) & (relt < half_sz)
        cnt = append(prec, cnt,
                     (idx_base + _PER_TEC - 16 + lane) | (relt << 18), mt)

        cnt_b[pl.ds(0, 16)] = lane * 0 + cnt
        pltpu.sync_copy(cnt_b, cnts_sp.at[pl.ds(s * 16, 16)])
        nblk = (cnt + (_BLK - 1)) // _BLK

        def pub(b, carry):
            pltpu.sync_copy(prec.at[pl.ds(b * _BLK, _BLK)],
                            mail.at[s, pl.ds(b * _BLK, _BLK)])
            return carry

        lax.fori_loop(0, nblk, pub, jnp.int32(0))
        plsc.subcore_barrier()

        pltpu.make_async_copy(cand_hbm.at[pl.ds(gbase, own_sz)],
                              acc.at[pl.ds(0, own_sz)], sem2).wait()
        pltpu.sync_copy(cnts_sp, cnts_v)

        def con_q(q, ocnt):
            cq = cnts_v[pl.ds(q * 16, 16)][0]
            nbq = (cq + (_BLK - 1)) // _BLK

            def con_b(b, ocnt):
                pltpu.sync_copy(mail.at[q, pl.ds(b * _BLK, _BLK)], blk_v)
                remaining = cq - b * _BLK
                nvr = (jnp.minimum(remaining, _BLK) + 15) // 16

                def filt(v, ocnt, off):
                    w = blk_v[pl.ds(v * 16, 16)]
                    rl = (w >> 18) - own_lo
                    vm = ((rl >= 0) & (rl < own_sz)
                          & (b * _BLK + v * 16 + lane < cq))
                    comb2 = (w & _PMASK) | (rl << 18)
                    pcv = plsc.all_reduce_population_count(vm)
                    plsc.store_compressed(stage_v.at[pl.ds(off, 16)],
                                          comb2, mask=vm)
                    w2 = stage_v[pl.ds(off, 16)]
                    plsc.store_scatter(orec, [ocnt + lane], w2,
                                       mask=lane < pcv)
                    return ocnt + pcv[0]

                def f_v2(u, ocnt):
                    ocnt = filt(2 * u, ocnt, 0)
                    return filt(2 * u + 1, ocnt, 16)

                ocnt = lax.fori_loop(0, nvr // 2, f_v2, ocnt)
                ocnt = lax.cond(nvr % 2 == 1,
                                lambda o: filt(nvr - 1, o, 0),
                                lambda o: o, ocnt)
                nfull = ocnt // _BATCH
                apply_batches(nfull)
                for t in range(_BATCH // 16):
                    w = orec[pl.ds(nfull * _BATCH + t * 16, 16)]
                    orec[pl.ds(t * 16, 16)] = w
                return ocnt - nfull * _BATCH

            return lax.fori_loop(0, nbq, con_b, ocnt)

        ocnt = lax.fori_loop(0, _NTEC, con_q, jnp.int32(0))

        for t in range(_BATCH // 16):
            dest = ocnt + t * 16 + lane
            padv = ((idx_base + t * 16 + lane)
                    | ((own_sz + (lane & 7)) << 18))
            plsc.store_scatter(orec, [dest], padv)
        apply_batches((ocnt + (_BATCH - 1)) // _BATCH)
        plsc.subcore_barrier()

        pltpu.sync_copy(acc.at[pl.ds(0, own_sz)],
                        out_hbm.at[pl.ds(gbase, own_sz)])
        plsc.subcore_barrier()

    def full_pass(p, carry):
        run_pass(p * _PASS, _HALF, _OWN)
        return carry

    lax.fori_loop(0, _NPF, full_pass, jnp.int32(0))
    run_pass(_NPF * _PASS, _LHALF, _LOWN)


def kernel(candidate_rep, graph_rep, graph_sizes, put_indices, Wu, bu, Ws, bs):
    upd = _scored_update(candidate_rep, graph_rep, Wu, bu, Ws, bs)
    out = _scatter_add(candidate_rep, put_indices, upd)
    return out, graph_rep


# speedup vs baseline: 2.6874x; 1.0942x over previous
import functools

import jax
import jax.numpy as jnp
from jax import lax
from jax.experimental import pallas as pl
from jax.experimental.pallas import tpu as pltpu
from jax.experimental.pallas import tpu_sc as plsc

N = 160000
ENC = 128
D_IN = 3 * ENC


_ROWS = 8000
_GRID = N // _ROWS
_WCOLS = 256


def _scored_update_body(cand_ref, graph_ref, w1_ref, w2_ref, b_ref, out_ref):
    x1 = cand_ref[...].astype(jnp.bfloat16)
    x2 = graph_ref[...].astype(jnp.bfloat16)
    y = (jnp.dot(x1, w1_ref[...], preferred_element_type=jnp.float32)
         + jnp.dot(x2, w2_ref[...], preferred_element_type=jnp.float32)
         + b_ref[...])
    u = y[:, :ENC]
    s = y[:, ENC:ENC + 1]
    out_ref[...] = u * s


def _scored_update(candidate_rep, graph_rep, Wu, bu, Ws, bs):
    w = jnp.zeros((D_IN, _WCOLS), jnp.float32)
    w = w.at[:, :ENC].set(Wu).at[:, ENC].set(Ws[:, 0])
    w = w.astype(jnp.bfloat16)
    b = jnp.zeros((1, _WCOLS), jnp.float32)
    b = b.at[0, :ENC].set(bu).at[0, ENC].set(bs[0])
    return pl.pallas_call(
        _scored_update_body,
        grid=(_GRID,),
        in_specs=[
            pl.BlockSpec((_ROWS, ENC), lambda i: (i, 0)),
            pl.BlockSpec((_ROWS, 2 * ENC), lambda i: (i, 0)),
            pl.BlockSpec((ENC, _WCOLS), lambda i: (0, 0)),
            pl.BlockSpec((2 * ENC, _WCOLS), lambda i: (0, 0)),
            pl.BlockSpec((1, _WCOLS), lambda i: (0, 0)),
        ],
        out_specs=pl.BlockSpec((_ROWS, ENC), lambda i: (i, 0)),
        out_shape=jax.ShapeDtypeStruct((N, ENC), jnp.float32),
    )(candidate_rep, graph_rep, w[:ENC], w[ENC:], b)


_NTEC = 16
_PASS = 16384
_HALF = _PASS // 2
_OWN = _HALF // _NTEC
_NPF = 9
_LWIN = N - _NPF * _PASS
_LHALF = _LWIN // 2
_LOWN = _LHALF // _NTEC
_GUARDR = 8
_PER_TEC = N // _NTEC
_SCAN_IT = _PER_TEC // 16
_BATCH = 128
_BLK = 2048
_PBUF = 5 * _BLK
_OBUF = _BLK + 3 * _BATCH
_PMASK = (1 << 18) - 1


@functools.partial(
    pl.kernel,
    out_type=jax.ShapeDtypeStruct((N, ENC), jnp.float32),
    mesh=plsc.VectorSubcoreMesh(core_axis_name="c", subcore_axis_name="s"),
    compiler_params=pltpu.CompilerParams(needs_layout_passes=False),
    scratch_types=[
        pltpu.VMEM((_PER_TEC,), jnp.int32),
        pltpu.VMEM((_PBUF,), jnp.int32),
        pltpu.VMEM((_OBUF,), jnp.int32),
        pltpu.VMEM((_BLK,), jnp.int32),
        pltpu.VMEM((256,), jnp.int32),
        pltpu.VMEM((16,), jnp.int32),
        pltpu.VMEM((32,), jnp.int32),
        pltpu.VMEM((64,), jnp.int32),
        pltpu.VMEM((64,), jnp.int32),
        pltpu.VMEM((_BATCH, ENC), jnp.float32),
        pltpu.VMEM((_OWN + _GUARDR, ENC), jnp.float32),
        pltpu.VMEM_SHARED((_NTEC, _PBUF), jnp.int32),
        pltpu.VMEM_SHARED((256,), jnp.int32),
        pltpu.SemaphoreType.DMA,
        pltpu.SemaphoreType.DMA,
    ],
)
def _scatter_add(cand_hbm, idx_hbm, upd_hbm, out_hbm,
                 idx_v, prec, orec, blk_v, cnts_v, cnt_b, stage_v, pos_b0,
                 pos_b1, rows_v, acc, mail, cnts_sp, sem, sem2):
    c = lax.axis_index("c")
    s = lax.axis_index("s")
    lane = lax.iota(jnp.int32, 16)
    idx_base = s * _PER_TEC

    pltpu.sync_copy(idx_hbm.at[pl.ds(idx_base, _PER_TEC)], idx_v)

    def append(buf, cnt, comb, m):
        pcv = plsc.all_reduce_population_count(m)
        plsc.store_compressed(stage_v.at[pl.ds(0, 16)], comb, mask=m)
        w = stage_v[pl.ds(0, 16)]
        plsc.store_scatter(buf, [cnt + lane], w, mask=lane < pcv)
        return cnt + pcv[0]

    def decode(hb, pos_b):
        for t in range(4):
            w = orec[pl.ds(hb * 64 + t * 16, 16)]
            pos_b[pl.ds(t * 16, 16)] = w & _PMASK

    def start(pos_b, half):
        return pltpu.async_copy(upd_hbm.at[pos_b],
                                rows_v.at[pl.ds(half * 64, 64)], sem)

    def drain(half):
        pltpu.make_async_copy(upd_hbm.at[pos_b0],
                              rows_v.at[pl.ds(half * 64, 64)], sem).wait()

    def apply64(hb, half):
        def grp(g, carry2):
            w8 = orec[pl.ds(hb * 64 + g * 16, 16)]
            loc = w8 >> 18
            for j in range(16):
                lj = loc[j]
                for t in range(ENC // 16):
                    plsc.addupdate(
                        acc.at[lj, pl.ds(t * 16, 16)],
                        rows_v[half * 64 + g * 16 + j, pl.ds(t * 16, 16)])
            return carry2

        lax.fori_loop(0, 4, grp, jnp.int32(0))

    def apply_batches(npair):
        @pl.when(npair > 0)
        def _pipe():
            decode(0, pos_b0)
            start(pos_b0, 0)

            def pair(pb, carry):
                hb0 = 2 * pb
                decode(hb0 + 1, pos_b1)
                drain(0)
                start(pos_b1, 1)
                apply64(hb0, 0)

                @pl.when(pb + 1 < npair)
                def _pf():
                    decode(hb0 + 2, pos_b0)

                drain(1)

                @pl.when(pb + 1 < npair)
                def _st():
                    start(pos_b0, 0)

                apply64(hb0 + 1, 1)
                return carry

            lax.fori_loop(0, npair, pair, jnp.int32(0))

    def run_pass(win_base, half_sz, own_sz):
        sc_base = win_base + c * half_sz
        own_lo = s * own_sz
        gbase = sc_base + own_lo

        pltpu.async_copy(cand_hbm.at[pl.ds(gbase, own_sz)],
                         acc.at[pl.ds(0, own_sz)], sem2)

        def scan_body(i, cnt):
            iv0 = idx_v[pl.ds(i * 32, 16)]
            iv1 = idx_v[pl.ds(i * 32 + 16, 16)]
            rel0 = iv0 - sc_base
            rel1 = iv1 - sc_base
            m0 = (rel0 >= 0) & (rel0 < half_sz)
            m1 = (rel1 >= 0) & (rel1 < half_sz)
            comb0 = (idx_base + i * 32 + lane) | (rel0 << 18)
            comb1 = (idx_base + i * 32 + 16 + lane) | (rel1 << 18)
            pcv0 = plsc.all_reduce_population_count(m0)
            pcv1 = plsc.all_reduce_population_count(m1)
            plsc.store_compressed(stage_v.at[pl.ds(0, 16)], comb0, mask=m0)
            plsc.store_compressed(stage_v.at[pl.ds(16, 16)], comb1, mask=m1)
            w0 = stage_v[pl.ds(0, 16)]
            w1 = stage_v[pl.ds(16, 16)]
            cnt1 = cnt + pcv0[0]
            plsc.store_scatter(prec, [cnt + lane], w0, mask=lane < pcv0)
            plsc.store_scatter(prec, [cnt1 + lane], w1, mask=lane < pcv1)
            return cnt1 + pcv1[0]

        cnt = lax.fori_loop(0, _SCAN_IT // 2, scan_body, jnp.int32(0))
        ivt = idx_v[pl.ds(_PER_TEC - 16, 16)]
        relt = ivt - sc_base
        mt = (relt >= 0) & (relt < half_sz)
        cnt = append(prec, cnt,
                     (idx_base + _PER_TEC - 16 + lane) | (relt << 18), mt)

        cnt_b[pl.ds(0, 16)] = lane * 0 + cnt
        pltpu.sync_copy(cnt_b, cnts_sp.at[pl.ds(s * 16, 16)])
        nblk = (cnt + (_BLK - 1)) // _BLK

        def pub(b, carry):
            pltpu.sync_copy(prec.at[pl.ds(b * _BLK, _BLK)],
                            mail.at[s, pl.ds(b * _BLK, _BLK)])
            return carry

        lax.fori_loop(0, nblk, pub, jnp.int32(0))
        plsc.subcore_barrier()

        pltpu.make_async_copy(cand_hbm.at[pl.ds(gbase, own_sz)],
                              acc.at[pl.ds(0, own_sz)], sem2).wait()
        pltpu.sync_copy(cnts_sp, cnts_v)

        def con_q(q, ocnt):
            cq = cnts_v[pl.ds(q * 16, 16)][0]
            nbq = (cq + (_BLK - 1)) // _BLK

            def con_b(b, ocnt):
                pltpu.sync_copy(mail.at[q, pl.ds(b * _BLK, _BLK)], blk_v)
                remaining = cq - b * _BLK
                nvr = (jnp.minimum(remaining, _BLK) + 15) // 16

                def filt(v, ocnt, off):
                    w = blk_v[pl.ds(v * 16, 16)]
                    rl = (w >> 18) - own_lo
                    vm = ((rl >= 0) & (rl < own_sz)
                          & (b * _BLK + v * 16 + lane < cq))
                    comb2 = (w & _PMASK) | (rl << 18)
                    pcv = plsc.all_reduce_population_count(vm)
                    plsc.store_compressed(stage_v.at[pl.ds(off, 16)],
                                          comb2, mask=vm)
                    w2 = stage_v[pl.ds(off, 16)]
                    plsc.store_scatter(orec, [ocnt + lane], w2,
                                       mask=lane < pcv)
                    return ocnt + pcv[0]

                def f_v2(u, ocnt):
                    ocnt = filt(2 * u, ocnt, 0)
                    return filt(2 * u + 1, ocnt, 16)

                ocnt = lax.fori_loop(0, nvr // 2, f_v2, ocnt)
                ocnt = lax.cond(nvr % 2 == 1,
                                lambda o: filt(nvr - 1, o, 0),
                                lambda o: o, ocnt)
                nfull = ocnt // _BATCH
                apply_batches(nfull)
                for t in range(_BATCH // 16):
                    w = orec[pl.ds(nfull * _BATCH + t * 16, 16)]
                    orec[pl.ds(t * 16, 16)] = w
                return ocnt - nfull * _BATCH

            return lax.fori_loop(0, nbq, con_b, ocnt)

        ocnt = lax.fori_loop(0, _NTEC, con_q, jnp.int32(0))

        for t in range(_BATCH // 16):
            dest = ocnt + t * 16 + lane
            padv = ((idx_base + t * 16 + lane)
                    | ((own_sz + (lane & 7)) << 18))
            plsc.store_scatter(orec, [dest], padv)
        apply_batches((ocnt + (_BATCH - 1)) // _BATCH)
        plsc.subcore_barrier()

        pltpu.sync_copy(acc.at[pl.ds(0, own_sz)],
                        out_hbm.at[pl.ds(gbase, own_sz)])
        plsc.subcore_barrier()

    def full_pass(p, carry):
        run_pass(p * _PASS, _HALF, _OWN)
        return carry

    lax.fori_loop(0, _NPF, full_pass, jnp.int32(0))
    run_pass(_NPF * _PASS, _LHALF, _LOWN)


def kernel(candidate_rep, graph_rep, graph_sizes, put_indices, Wu, bu, Ws, bs):
    upd = _scored_update(candidate_rep, graph_rep, Wu, bu, Ws, bs)
    out = _scatter_add(candidate_rep, put_indices, upd)
    return out, graph_rep
